# Initial kernel scaffold; baseline (speedup 1.0000x reference)
import jax
import jax.numpy as jnp
from jax.experimental import pallas as pl


def kernel(x, edge_index, mlp0_W1, mlp0_b1, mlp0_bn_g, mlp0_bn_b, mlp0_W2, mlp0_b2, mlp1_W1, mlp1_b1, mlp1_bn_g, mlp1_bn_b, mlp1_W2, mlp1_b2, bn0_g, bn0_b, bn1_g, bn1_b, pred0_W, pred0_b, pred1_W, pred1_b, pred2_W, pred2_b):
    raise NotImplementedError("write your pallas kernel here")


# speedup vs baseline: 4.3226x; 4.3226x over previous
import functools

import jax
import jax.numpy as jnp
from jax import lax
from jax.experimental import pallas as pl
from jax.experimental.pallas import tpu as pltpu
from jax.experimental.pallas import tpu_sc as plsc

N_NODES = 10000
D = 128
NC = 2
NS = 16
NW = NC * NS
CHUNK = 128
N_PAD = 10240
ROWS_PER_TILE = N_PAD // NS
TABLE_PAD = 10008


def _make_sc_scatter(K):
    mesh = plsc.VectorSubcoreMesh(core_axis_name="c", subcore_axis_name="s")

    @functools.partial(
        pl.kernel,
        mesh=mesh,
        out_type=jax.ShapeDtypeStruct((NC, N_PAD, D), jnp.float32),
        scratch_types=[
            pltpu.VMEM((K, CHUNK), jnp.int32),
            pltpu.VMEM((K, CHUNK), jnp.int32),
            pltpu.VMEM((CHUNK, D), jnp.float32),
            pltpu.VMEM_SHARED((N_PAD, D), jnp.float32),
            pltpu.SemaphoreType.DMA,
        ],
    )
    def sc_scatter(h_hbm, row_hbm, col_hbm, out_hbm, row_v, col_v, gbuf, agg_sh, sem):
        c = lax.axis_index("c")
        s = lax.axis_index("s")
        w = s * NC + c

        zero16 = jnp.zeros((16,), jnp.float32)

        def zrow(r, carry):
            for cc in range(D // 16):
                gbuf[r, pl.ds(cc * 16, 16)] = zero16
            return carry

        lax.fori_loop(0, CHUNK, zrow, 0)
        base = s * ROWS_PER_TILE
        for j in range(ROWS_PER_TILE // CHUNK):
            pltpu.sync_copy(gbuf, agg_sh.at[pl.ds(base + j * CHUNK, CHUNK)])
        plsc.subcore_barrier()

        pltpu.sync_copy(row_hbm.at[w], row_v)
        pltpu.sync_copy(col_hbm.at[w], col_v)

        def body(k, carry):
            pltpu.async_copy(h_hbm.at[col_v.at[k]], gbuf, sem).wait()
            pltpu.sync_copy(gbuf, agg_sh.at[row_v.at[k]], add=True)
            return carry

        lax.fori_loop(0, K, body, 0)
        plsc.subcore_barrier()

        for j in range(ROWS_PER_TILE // CHUNK):
            pltpu.sync_copy(agg_sh.at[pl.ds(base + j * CHUNK, CHUNK)], gbuf)
            pltpu.sync_copy(gbuf, out_hbm.at[c, pl.ds(base + j * CHUNK, CHUNK)])

    return sc_scatter


def _bn_relu(y, g, b):
    m = jnp.mean(y, axis=0, keepdims=True)
    v = jnp.mean((y - m) ** 2, axis=0, keepdims=True)
    return jnp.maximum((y - m) * lax.rsqrt(v + 1e-5) * g + b, 0.0)


def _dense_layer_body(h_ref, a0_ref, a1_ref, W1_ref, b1_ref, g1_ref, bb1_ref,
                      W2_ref, b2_ref, g2_ref, bb2_ref, out_ref):
    t = h_ref[...] + a0_ref[...] + a1_ref[...]
    y = jnp.dot(t, W1_ref[...], preferred_element_type=jnp.float32) + b1_ref[...]
    y = _bn_relu(y, g1_ref[...], bb1_ref[...])
    z = jnp.dot(y, W2_ref[...], preferred_element_type=jnp.float32) + b2_ref[...]
    out_ref[...] = _bn_relu(z, g2_ref[...], bb2_ref[...])


def _dense_pred_body(h_ref, a0_ref, a1_ref, W1_ref, b1_ref, g1_ref, bb1_ref,
                     W2_ref, b2_ref, g2_ref, bb2_ref,
                     h0_ref, P0_ref, P1_ref, P2_ref, pb_ref, out_ref):
    t = h_ref[...] + a0_ref[...] + a1_ref[...]
    y = jnp.dot(t, W1_ref[...], preferred_element_type=jnp.float32) + b1_ref[...]
    y = _bn_relu(y, g1_ref[...], bb1_ref[...])
    z = jnp.dot(y, W2_ref[...], preferred_element_type=jnp.float32) + b2_ref[...]
    h2 = _bn_relu(z, g2_ref[...], bb2_ref[...])
    out_ref[...] = (jnp.dot(h0_ref[...], P0_ref[...], preferred_element_type=jnp.float32)
                    + jnp.dot(h_ref[...], P1_ref[...], preferred_element_type=jnp.float32)
                    + jnp.dot(h2, P2_ref[...], preferred_element_type=jnp.float32)
                    + pb_ref[...])


_dense_layer = pl.pallas_call(
    _dense_layer_body,
    out_shape=jax.ShapeDtypeStruct((N_NODES, D), jnp.float32),
)

_dense_pred = pl.pallas_call(
    _dense_pred_body,
    out_shape=jax.ShapeDtypeStruct((N_NODES, 32), jnp.float32),
)


def kernel(x, edge_index,
           mlp0_W1, mlp0_b1, mlp0_bn_g, mlp0_bn_b, mlp0_W2, mlp0_b2,
           mlp1_W1, mlp1_b1, mlp1_bn_g, mlp1_bn_b, mlp1_W2, mlp1_b2,
           bn0_g, bn0_b, bn1_g, bn1_b,
           pred0_W, pred0_b, pred1_W, pred1_b, pred2_W, pred2_b):
    row = edge_index[0]
    col = edge_index[1]
    E = row.shape[0]
    K = -(-E // (NW * CHUNK))
    E_pad = K * NW * CHUNK
    pad = E_pad - E
    row_p = jnp.concatenate([row, jnp.full((pad,), N_PAD - 1, jnp.int32)]).reshape(NW, K, CHUNK)
    col_p = jnp.concatenate([col, jnp.full((pad,), N_NODES, jnp.int32)]).reshape(NW, K, CHUNK)

    sc_scatter = _make_sc_scatter(K)
    zrows = jnp.zeros((TABLE_PAD - N_NODES, D), jnp.float32)

    def r2(v):
        return v.reshape(1, -1)

    h0 = x
    parts0 = sc_scatter(jnp.concatenate([h0, zrows], axis=0), row_p, col_p)
    h1 = _dense_layer(h0, parts0[0, :N_NODES], parts0[1, :N_NODES],
                      mlp0_W1, r2(mlp0_b1), r2(mlp0_bn_g), r2(mlp0_bn_b),
                      mlp0_W2, r2(mlp0_b2), r2(bn0_g), r2(bn0_b))
    parts1 = sc_scatter(jnp.concatenate([h1, zrows], axis=0), row_p, col_p)
    out = _dense_pred(h1, parts1[0, :N_NODES], parts1[1, :N_NODES],
                      mlp1_W1, r2(mlp1_b1), r2(mlp1_bn_g), r2(mlp1_bn_b),
                      mlp1_W2, r2(mlp1_b2), r2(bn1_g), r2(bn1_b),
                      h0, pred0_W, pred1_W, pred2_W,
                      r2(pred0_b + pred1_b + pred2_b))
    return out
